# baseline (speedup 1.0000x reference)
import jax
import jax.numpy as jnp
from jax import lax
from jax.experimental import pallas as pl
from jax.experimental.pallas import tpu as pltpu

_LANES = 128
_BR = 512
_SUB = 8


def _pass_body(cv_ref, x_ref, y_ref, stat_ref, accd_ref, accv_ref, acci_ref):
    i = pl.program_id(0)
    n = pl.num_programs(0)

    shp = (_SUB, _LANES)
    lane = lax.broadcasted_iota(jnp.int32, shp, 1)
    sub = lax.broadcasted_iota(jnp.int32, shp, 0)
    odd = (lane & 1) == 1
    cvb = jnp.where(odd, cv_ref[1], cv_ref[0])
    idx0 = (sub << 6) | (lane >> 1)

    pos = jnp.float32(jnp.inf)
    neg = jnp.float32(-jnp.inf)

    @pl.when(i == 0)
    def _init():
        accd_ref[...] = jnp.full(shp, pos)
        accv_ref[...] = jnp.full(shp, neg)
        acci_ref[...] = jnp.zeros(shp, jnp.int32)

    accd = accd_ref[...]
    accv = accv_ref[...]
    acci = acci_ref[...]

    base = i * (_BR * 64)
    for j in range(_BR // _SUB):
        x = x_ref[j * _SUB:(j + 1) * _SUB, :]
        y_ref[j * _SUB:(j + 1) * _SUB, :] = x
        d = x - cvb
        pm = jnp.maximum(d, pltpu.roll(d, _LANES - 1, 1))
        accd = jnp.minimum(accd, pm)
        gt = x > accv
        accv = jnp.where(gt, x, accv)
        acci = jnp.where(gt, idx0 + (base + j * _SUB * 64), acci)

    accd_ref[...] = accd
    accv_ref[...] = accv
    acci_ref[...] = acci

    @pl.when(i == n - 1)
    def _fin():
        big = jnp.int32(2147483647)
        dmin = jnp.min(jnp.where(odd, pos, accd))
        stat_ref[0] = jnp.where(dmin <= 0.0, jnp.int32(1), jnp.int32(0))
        v0 = jnp.where(odd, neg, accv)
        v1 = jnp.where(odd, accv, neg)
        m0 = jnp.max(v0)
        m1 = jnp.max(v1)
        i0 = jnp.min(jnp.where(v0 == m0, acci, big))
        i1 = jnp.min(jnp.where(v1 == m1, acci, big))
        stat_ref[1] = jnp.maximum(i0, i1)


def _scatter_body(s_ref, x_ref, cv_ref, y_ref):
    u = s_ref[0]
    aug = s_ref[1]
    x = x_ref[...]
    lane = lax.broadcasted_iota(jnp.int32, x.shape, 1)
    sub = lax.broadcasted_iota(jnp.int32, x.shape, 0)
    rs = (u // 64) % 8
    lp = (u % 64) * 2
    sel = (sub == rs) & ((lane == lp) | (lane == lp + 1)) & (aug != 0)
    val = jnp.where(lane == lp, cv_ref[0], cv_ref[1])
    y_ref[...] = jnp.where(sel, val, x)


def kernel(filter_by, current_values, augment):
    f, c = filter_by.shape
    r = (f * c) // _LANES
    x = filter_by.reshape(r, _LANES)

    y, stats = pl.pallas_call(
        _pass_body,
        grid=(r // _BR,),
        in_specs=[
            pl.BlockSpec(memory_space=pltpu.SMEM),
            pl.BlockSpec((_BR, _LANES), lambda i: (i, 0)),
        ],
        out_specs=[
            pl.BlockSpec((_BR, _LANES), lambda i: (i, 0)),
            pl.BlockSpec(memory_space=pltpu.SMEM),
        ],
        out_shape=[
            jax.ShapeDtypeStruct((r, _LANES), jnp.float32),
            jax.ShapeDtypeStruct((2,), jnp.int32),
        ],
        scratch_shapes=[
            pltpu.VMEM((_SUB, _LANES), jnp.float32),
            pltpu.VMEM((_SUB, _LANES), jnp.float32),
            pltpu.VMEM((_SUB, _LANES), jnp.int32),
        ],
    )(current_values, x)

    scal = jnp.stack([stats[1], augment.astype(jnp.int32)])
    y2 = pl.pallas_call(
        _scatter_body,
        grid_spec=pltpu.PrefetchScalarGridSpec(
            num_scalar_prefetch=1,
            grid=(1,),
            in_specs=[
                pl.BlockSpec((8, _LANES), lambda i, s: (s[0] // 512, 0)),
                pl.BlockSpec(memory_space=pltpu.SMEM),
            ],
            out_specs=pl.BlockSpec((8, _LANES), lambda i, s: (s[0] // 512, 0)),
        ),
        out_shape=jax.ShapeDtypeStruct((r, _LANES), jnp.float32),
        input_output_aliases={1: 0},
    )(scal, y, current_values)

    return stats[0] == 0, y2.reshape(f, c)


# speedup vs baseline: 91.3725x; 91.3725x over previous
import jax
import jax.numpy as jnp
from jax import lax
from jax.experimental import pallas as pl
from jax.experimental.pallas import tpu as pltpu

_LANES = 128
_BR = 512
_SUB = 8


def _pass_body(cv_ref, x_ref, y_ref, stat_ref, accd_ref, accv_ref, acci_ref):
    i = pl.program_id(0)
    n = pl.num_programs(0)

    shp = (_SUB, _LANES)
    lane = lax.broadcasted_iota(jnp.int32, shp, 1)
    sub = lax.broadcasted_iota(jnp.int32, shp, 0)
    oddsub = (sub & 1) == 1
    cvb = jnp.where(oddsub, cv_ref[1], cv_ref[0])
    idx0 = ((sub >> 1) << 7) | lane

    pos = jnp.float32(jnp.inf)
    neg = jnp.float32(-jnp.inf)

    @pl.when(i == 0)
    def _init():
        accd_ref[...] = jnp.full(shp, pos)
        accv_ref[...] = jnp.full(shp, neg)
        acci_ref[...] = jnp.zeros(shp, jnp.int32)

    accd = accd_ref[...]
    accv = accv_ref[...]
    acci = acci_ref[...]

    base = i * (_BR // _SUB) * 512
    for j in range(_BR // _SUB):
        x = x_ref[j * _SUB:(j + 1) * _SUB, :]
        y_ref[j * _SUB:(j + 1) * _SUB, :] = x
        d = x - cvb
        pm = jnp.maximum(d, pltpu.roll(d, _SUB - 1, 0))
        accd = jnp.minimum(accd, pm)
        gt = x > accv
        accv = jnp.where(gt, x, accv)
        acci = jnp.where(gt, idx0 + (base + j * 512), acci)

    accd_ref[...] = accd
    accv_ref[...] = accv
    acci_ref[...] = acci

    @pl.when(i == n - 1)
    def _fin():
        big = jnp.int32(2147483647)
        dmin = jnp.min(jnp.where(oddsub, pos, accd))
        stat_ref[0] = jnp.where(dmin <= 0.0, jnp.int32(1), jnp.int32(0))
        v0 = jnp.where(oddsub, neg, accv)
        v1 = jnp.where(oddsub, accv, neg)
        m0 = jnp.max(v0)
        m1 = jnp.max(v1)
        i0 = jnp.min(jnp.where(v0 == m0, acci, big))
        i1 = jnp.min(jnp.where(v1 == m1, acci, big))
        stat_ref[1] = jnp.maximum(i0, i1)


def _scatter_body(s_ref, x_ref, cv_ref, y_ref):
    u = s_ref[0]
    aug = s_ref[1]
    x = x_ref[...]
    lane = lax.broadcasted_iota(jnp.int32, x.shape, 1)
    sub = lax.broadcasted_iota(jnp.int32, x.shape, 0)
    lp = u % 128
    s0 = ((u // 128) % 4) * 2
    sel = (lane == lp) & ((sub == s0) | (sub == s0 + 1)) & (aug != 0)
    val = jnp.where(sub == s0, cv_ref[0], cv_ref[1])
    y_ref[...] = jnp.where(sel, val, x)


def kernel(filter_by, current_values, augment):
    f, c = filter_by.shape
    r = (f * c) // _LANES
    w = (filter_by.reshape(r // 8, 4, _LANES, 2)
         .transpose(0, 1, 3, 2)
         .reshape(r, _LANES))

    y, stats = pl.pallas_call(
        _pass_body,
        grid=(r // _BR,),
        in_specs=[
            pl.BlockSpec(memory_space=pltpu.SMEM),
            pl.BlockSpec((_BR, _LANES), lambda i: (i, 0)),
        ],
        out_specs=[
            pl.BlockSpec((_BR, _LANES), lambda i: (i, 0)),
            pl.BlockSpec(memory_space=pltpu.SMEM),
        ],
        out_shape=[
            jax.ShapeDtypeStruct((r, _LANES), jnp.float32),
            jax.ShapeDtypeStruct((2,), jnp.int32),
        ],
        scratch_shapes=[
            pltpu.VMEM((_SUB, _LANES), jnp.float32),
            pltpu.VMEM((_SUB, _LANES), jnp.float32),
            pltpu.VMEM((_SUB, _LANES), jnp.int32),
        ],
    )(current_values, w)

    scal = jnp.stack([stats[1], augment.astype(jnp.int32)])
    y2 = pl.pallas_call(
        _scatter_body,
        grid_spec=pltpu.PrefetchScalarGridSpec(
            num_scalar_prefetch=1,
            grid=(1,),
            in_specs=[
                pl.BlockSpec((8, _LANES), lambda i, s: (s[0] // 512, 0)),
                pl.BlockSpec(memory_space=pltpu.SMEM),
            ],
            out_specs=pl.BlockSpec((8, _LANES), lambda i, s: (s[0] // 512, 0)),
        ),
        out_shape=jax.ShapeDtypeStruct((r, _LANES), jnp.float32),
        input_output_aliases={1: 0},
    )(scal, y, current_values)

    out = (y2.reshape(r // 8, 4, 2, _LANES)
           .transpose(0, 1, 3, 2)
           .reshape(f, c))
    return stats[0] == 0, out


# speedup vs baseline: 139.4432x; 1.5261x over previous
import jax
import jax.numpy as jnp
from jax import lax
from jax.experimental import pallas as pl
from jax.experimental.pallas import tpu as pltpu

_LANES = 128
_BR = 1024
_SUB = 8


def _pass_body(cv_ref, x_ref, y_ref, stat_ref, accd_ref, accv_ref, acci_ref):
    i = pl.program_id(0)
    n = pl.num_programs(0)

    shp = (_SUB, _LANES)
    lane = lax.broadcasted_iota(jnp.int32, shp, 1)
    sub = lax.broadcasted_iota(jnp.int32, shp, 0)
    oddsub = (sub & 1) == 1
    cvb = jnp.where(oddsub, cv_ref[1], cv_ref[0])
    idx0 = ((sub >> 1) << 7) | lane

    pos = jnp.float32(jnp.inf)
    neg = jnp.float32(-jnp.inf)

    @pl.when(i == 0)
    def _init():
        accd_ref[...] = jnp.full(shp, pos)
        accv_ref[...] = jnp.full(shp, neg)
        acci_ref[...] = jnp.zeros(shp, jnp.int32)

    accd = accd_ref[...]
    accv = accv_ref[...]
    acci = acci_ref[...]

    base = i * (_BR // _SUB) * 512
    for j in range(_BR // _SUB):
        x = x_ref[j * _SUB:(j + 1) * _SUB, :]
        y_ref[j * _SUB:(j + 1) * _SUB, :] = x
        d = x - cvb
        pm = jnp.maximum(d, pltpu.roll(d, _SUB - 1, 0))
        accd = jnp.minimum(accd, pm)
        gt = x > accv
        accv = jnp.where(gt, x, accv)
        acci = jnp.where(gt, idx0 + (base + j * 512), acci)

    accd_ref[...] = accd
    accv_ref[...] = accv
    acci_ref[...] = acci

    @pl.when(i == n - 1)
    def _fin():
        big = jnp.int32(2147483647)
        dmin = jnp.min(jnp.where(oddsub, pos, accd))
        stat_ref[0] = jnp.where(dmin <= 0.0, jnp.int32(1), jnp.int32(0))
        v0 = jnp.where(oddsub, neg, accv)
        v1 = jnp.where(oddsub, accv, neg)
        m0 = jnp.max(v0)
        m1 = jnp.max(v1)
        i0 = jnp.min(jnp.where(v0 == m0, acci, big))
        i1 = jnp.min(jnp.where(v1 == m1, acci, big))
        stat_ref[1] = jnp.maximum(i0, i1)


def _scatter_body(s_ref, x_ref, cv_ref, y_ref):
    u = s_ref[0]
    aug = s_ref[1]
    x = x_ref[...]
    lane = lax.broadcasted_iota(jnp.int32, x.shape, 1)
    sub = lax.broadcasted_iota(jnp.int32, x.shape, 0)
    lp = u % 128
    s0 = ((u // 128) % 4) * 2
    sel = (lane == lp) & ((sub == s0) | (sub == s0 + 1)) & (aug != 0)
    val = jnp.where(sub == s0, cv_ref[0], cv_ref[1])
    y_ref[...] = jnp.where(sel, val, x)


def kernel(filter_by, current_values, augment):
    f, c = filter_by.shape
    r = (f * c) // _LANES
    w = (filter_by.reshape(r // 8, 4, _LANES, 2)
         .transpose(0, 1, 3, 2)
         .reshape(r, _LANES))

    y, stats = pl.pallas_call(
        _pass_body,
        grid=(r // _BR,),
        in_specs=[
            pl.BlockSpec(memory_space=pltpu.SMEM),
            pl.BlockSpec((_BR, _LANES), lambda i: (i, 0)),
        ],
        out_specs=[
            pl.BlockSpec((_BR, _LANES), lambda i: (i, 0)),
            pl.BlockSpec(memory_space=pltpu.SMEM),
        ],
        out_shape=[
            jax.ShapeDtypeStruct((r, _LANES), jnp.float32),
            jax.ShapeDtypeStruct((2,), jnp.int32),
        ],
        scratch_shapes=[
            pltpu.VMEM((_SUB, _LANES), jnp.float32),
            pltpu.VMEM((_SUB, _LANES), jnp.float32),
            pltpu.VMEM((_SUB, _LANES), jnp.int32),
        ],
    )(current_values, w)

    scal = jnp.stack([stats[1], augment.astype(jnp.int32)])
    y2 = pl.pallas_call(
        _scatter_body,
        grid_spec=pltpu.PrefetchScalarGridSpec(
            num_scalar_prefetch=1,
            grid=(1,),
            in_specs=[
                pl.BlockSpec((8, _LANES), lambda i, s: (s[0] // 512, 0)),
                pl.BlockSpec(memory_space=pltpu.SMEM),
            ],
            out_specs=pl.BlockSpec((8, _LANES), lambda i, s: (s[0] // 512, 0)),
        ),
        out_shape=jax.ShapeDtypeStruct((r, _LANES), jnp.float32),
        input_output_aliases={1: 0},
    )(scal, y, current_values)

    out = (y2.reshape(r // 8, 4, 2, _LANES)
           .transpose(0, 1, 3, 2)
           .reshape(f, c))
    return stats[0] == 0, out


# speedup vs baseline: 182.8600x; 1.3114x over previous
import jax
import jax.numpy as jnp
from jax import lax
from jax.experimental import pallas as pl
from jax.experimental.pallas import tpu as pltpu

_LANES = 128
_BR = 2048
_SUB = 8


def _pass_body(cv_ref, x_ref, y_ref, stat_ref, accd_ref, accv_ref, acci_ref):
    i = pl.program_id(0)
    n = pl.num_programs(0)

    shp = (_SUB, _LANES)
    lane = lax.broadcasted_iota(jnp.int32, shp, 1)
    sub = lax.broadcasted_iota(jnp.int32, shp, 0)
    oddsub = (sub & 1) == 1
    cvb = jnp.where(oddsub, cv_ref[1], cv_ref[0])
    idx0 = ((sub >> 1) << 7) | lane

    pos = jnp.float32(jnp.inf)
    neg = jnp.float32(-jnp.inf)

    @pl.when(i == 0)
    def _init():
        accd_ref[...] = jnp.full(shp, pos)
        accv_ref[...] = jnp.full(shp, neg)
        acci_ref[...] = jnp.zeros(shp, jnp.int32)

    accd = accd_ref[...]
    accv = accv_ref[...]
    acci = acci_ref[...]

    base = i * (_BR // _SUB) * 512
    for j in range(_BR // _SUB):
        x = x_ref[j * _SUB:(j + 1) * _SUB, :]
        y_ref[j * _SUB:(j + 1) * _SUB, :] = x
        d = x - cvb
        pm = jnp.maximum(d, pltpu.roll(d, _SUB - 1, 0))
        accd = jnp.minimum(accd, pm)
        gt = x > accv
        accv = jnp.where(gt, x, accv)
        acci = jnp.where(gt, idx0 + (base + j * 512), acci)

    accd_ref[...] = accd
    accv_ref[...] = accv
    acci_ref[...] = acci

    @pl.when(i == n - 1)
    def _fin():
        big = jnp.int32(2147483647)
        dmin = jnp.min(jnp.where(oddsub, pos, accd))
        stat_ref[0] = jnp.where(dmin <= 0.0, jnp.int32(1), jnp.int32(0))
        v0 = jnp.where(oddsub, neg, accv)
        v1 = jnp.where(oddsub, accv, neg)
        m0 = jnp.max(v0)
        m1 = jnp.max(v1)
        i0 = jnp.min(jnp.where(v0 == m0, acci, big))
        i1 = jnp.min(jnp.where(v1 == m1, acci, big))
        stat_ref[1] = jnp.maximum(i0, i1)


def _scatter_body(s_ref, x_ref, cv_ref, y_ref):
    u = s_ref[0]
    aug = s_ref[1]
    x = x_ref[...]
    lane = lax.broadcasted_iota(jnp.int32, x.shape, 1)
    sub = lax.broadcasted_iota(jnp.int32, x.shape, 0)
    lp = u % 128
    s0 = ((u // 128) % 4) * 2
    sel = (lane == lp) & ((sub == s0) | (sub == s0 + 1)) & (aug != 0)
    val = jnp.where(sub == s0, cv_ref[0], cv_ref[1])
    y_ref[...] = jnp.where(sel, val, x)


def kernel(filter_by, current_values, augment):
    f, c = filter_by.shape
    r = (f * c) // _LANES
    w = (filter_by.reshape(r // 8, 4, _LANES, 2)
         .transpose(0, 1, 3, 2)
         .reshape(r, _LANES))

    y, stats = pl.pallas_call(
        _pass_body,
        grid=(r // _BR,),
        in_specs=[
            pl.BlockSpec(memory_space=pltpu.SMEM),
            pl.BlockSpec((_BR, _LANES), lambda i: (i, 0)),
        ],
        out_specs=[
            pl.BlockSpec((_BR, _LANES), lambda i: (i, 0)),
            pl.BlockSpec(memory_space=pltpu.SMEM),
        ],
        out_shape=[
            jax.ShapeDtypeStruct((r, _LANES), jnp.float32),
            jax.ShapeDtypeStruct((2,), jnp.int32),
        ],
        scratch_shapes=[
            pltpu.VMEM((_SUB, _LANES), jnp.float32),
            pltpu.VMEM((_SUB, _LANES), jnp.float32),
            pltpu.VMEM((_SUB, _LANES), jnp.int32),
        ],
    )(current_values, w)

    scal = jnp.stack([stats[1], augment.astype(jnp.int32)])
    y2 = pl.pallas_call(
        _scatter_body,
        grid_spec=pltpu.PrefetchScalarGridSpec(
            num_scalar_prefetch=1,
            grid=(1,),
            in_specs=[
                pl.BlockSpec((8, _LANES), lambda i, s: (s[0] // 512, 0)),
                pl.BlockSpec(memory_space=pltpu.SMEM),
            ],
            out_specs=pl.BlockSpec((8, _LANES), lambda i, s: (s[0] // 512, 0)),
        ),
        out_shape=jax.ShapeDtypeStruct((r, _LANES), jnp.float32),
        input_output_aliases={1: 0},
    )(scal, y, current_values)

    out = (y2.reshape(r // 8, 4, 2, _LANES)
           .transpose(0, 1, 3, 2)
           .reshape(f, c))
    return stats[0] == 0, out


# speedup vs baseline: 228.7225x; 1.2508x over previous
import jax
import jax.numpy as jnp
from jax import lax
from jax.experimental import pallas as pl
from jax.experimental.pallas import tpu as pltpu

_LANES = 128
_BR = 4096
_SUB = 8


def _pass_body(cv_ref, x_ref, y_ref, stat_ref, accd_ref, accv_ref, acci_ref):
    i = pl.program_id(0)
    n = pl.num_programs(0)

    shp = (_SUB, _LANES)
    lane = lax.broadcasted_iota(jnp.int32, shp, 1)
    sub = lax.broadcasted_iota(jnp.int32, shp, 0)
    oddsub = (sub & 1) == 1
    cvb = jnp.where(oddsub, cv_ref[1], cv_ref[0])
    idx0 = ((sub >> 1) << 7) | lane

    pos = jnp.float32(jnp.inf)
    neg = jnp.float32(-jnp.inf)

    @pl.when(i == 0)
    def _init():
        accd_ref[...] = jnp.full(shp, pos)
        accv_ref[...] = jnp.full(shp, neg)
        acci_ref[...] = jnp.zeros(shp, jnp.int32)

    accd = accd_ref[...]
    accv = accv_ref[...]
    acci = acci_ref[...]

    base = i * (_BR // _SUB) * 512
    for j in range(_BR // _SUB):
        x = x_ref[j * _SUB:(j + 1) * _SUB, :]
        y_ref[j * _SUB:(j + 1) * _SUB, :] = x
        d = x - cvb
        pm = jnp.maximum(d, pltpu.roll(d, _SUB - 1, 0))
        accd = jnp.minimum(accd, pm)
        gt = x > accv
        accv = jnp.where(gt, x, accv)
        acci = jnp.where(gt, idx0 + (base + j * 512), acci)

    accd_ref[...] = accd
    accv_ref[...] = accv
    acci_ref[...] = acci

    @pl.when(i == n - 1)
    def _fin():
        big = jnp.int32(2147483647)
        dmin = jnp.min(jnp.where(oddsub, pos, accd))
        stat_ref[0] = jnp.where(dmin <= 0.0, jnp.int32(1), jnp.int32(0))
        v0 = jnp.where(oddsub, neg, accv)
        v1 = jnp.where(oddsub, accv, neg)
        m0 = jnp.max(v0)
        m1 = jnp.max(v1)
        i0 = jnp.min(jnp.where(v0 == m0, acci, big))
        i1 = jnp.min(jnp.where(v1 == m1, acci, big))
        stat_ref[1] = jnp.maximum(i0, i1)


def _scatter_body(s_ref, x_ref, cv_ref, y_ref):
    u = s_ref[0]
    aug = s_ref[1]
    x = x_ref[...]
    lane = lax.broadcasted_iota(jnp.int32, x.shape, 1)
    sub = lax.broadcasted_iota(jnp.int32, x.shape, 0)
    lp = u % 128
    s0 = ((u // 128) % 4) * 2
    sel = (lane == lp) & ((sub == s0) | (sub == s0 + 1)) & (aug != 0)
    val = jnp.where(sub == s0, cv_ref[0], cv_ref[1])
    y_ref[...] = jnp.where(sel, val, x)


def kernel(filter_by, current_values, augment):
    f, c = filter_by.shape
    r = (f * c) // _LANES
    w = (filter_by.reshape(r // 8, 4, _LANES, 2)
         .transpose(0, 1, 3, 2)
         .reshape(r, _LANES))

    y, stats = pl.pallas_call(
        _pass_body,
        grid=(r // _BR,),
        in_specs=[
            pl.BlockSpec(memory_space=pltpu.SMEM),
            pl.BlockSpec((_BR, _LANES), lambda i: (i, 0)),
        ],
        out_specs=[
            pl.BlockSpec((_BR, _LANES), lambda i: (i, 0)),
            pl.BlockSpec(memory_space=pltpu.SMEM),
        ],
        out_shape=[
            jax.ShapeDtypeStruct((r, _LANES), jnp.float32),
            jax.ShapeDtypeStruct((2,), jnp.int32),
        ],
        scratch_shapes=[
            pltpu.VMEM((_SUB, _LANES), jnp.float32),
            pltpu.VMEM((_SUB, _LANES), jnp.float32),
            pltpu.VMEM((_SUB, _LANES), jnp.int32),
        ],
    )(current_values, w)

    scal = jnp.stack([stats[1], augment.astype(jnp.int32)])
    y2 = pl.pallas_call(
        _scatter_body,
        grid_spec=pltpu.PrefetchScalarGridSpec(
            num_scalar_prefetch=1,
            grid=(1,),
            in_specs=[
                pl.BlockSpec((8, _LANES), lambda i, s: (s[0] // 512, 0)),
                pl.BlockSpec(memory_space=pltpu.SMEM),
            ],
            out_specs=pl.BlockSpec((8, _LANES), lambda i, s: (s[0] // 512, 0)),
        ),
        out_shape=jax.ShapeDtypeStruct((r, _LANES), jnp.float32),
        input_output_aliases={1: 0},
    )(scal, y, current_values)

    out = (y2.reshape(r // 8, 4, 2, _LANES)
           .transpose(0, 1, 3, 2)
           .reshape(f, c))
    return stats[0] == 0, out


# speedup vs baseline: 251.9185x; 1.1014x over previous
import jax
import jax.numpy as jnp
from jax import lax
from jax.experimental import pallas as pl
from jax.experimental.pallas import tpu as pltpu

_LANES = 128
_BR = 8192
_SUB = 8


def _pass_body(cv_ref, x_ref, y_ref, stat_ref, accd_ref, accv_ref, acci_ref):
    i = pl.program_id(0)
    n = pl.num_programs(0)

    shp = (_SUB, _LANES)
    lane = lax.broadcasted_iota(jnp.int32, shp, 1)
    sub = lax.broadcasted_iota(jnp.int32, shp, 0)
    oddsub = (sub & 1) == 1
    cvb = jnp.where(oddsub, cv_ref[1], cv_ref[0])
    idx0 = ((sub >> 1) << 7) | lane

    pos = jnp.float32(jnp.inf)
    neg = jnp.float32(-jnp.inf)

    @pl.when(i == 0)
    def _init():
        accd_ref[...] = jnp.full(shp, pos)
        accv_ref[...] = jnp.full(shp, neg)
        acci_ref[...] = jnp.zeros(shp, jnp.int32)

    accd = accd_ref[...]
    accv = accv_ref[...]
    acci = acci_ref[...]

    base = i * (_BR // _SUB) * 512
    for j in range(_BR // _SUB):
        x = x_ref[j * _SUB:(j + 1) * _SUB, :]
        y_ref[j * _SUB:(j + 1) * _SUB, :] = x
        d = x - cvb
        pm = jnp.maximum(d, pltpu.roll(d, _SUB - 1, 0))
        accd = jnp.minimum(accd, pm)
        gt = x > accv
        accv = jnp.where(gt, x, accv)
        acci = jnp.where(gt, idx0 + (base + j * 512), acci)

    accd_ref[...] = accd
    accv_ref[...] = accv
    acci_ref[...] = acci

    @pl.when(i == n - 1)
    def _fin():
        big = jnp.int32(2147483647)
        dmin = jnp.min(jnp.where(oddsub, pos, accd))
        stat_ref[0] = jnp.where(dmin <= 0.0, jnp.int32(1), jnp.int32(0))
        v0 = jnp.where(oddsub, neg, accv)
        v1 = jnp.where(oddsub, accv, neg)
        m0 = jnp.max(v0)
        m1 = jnp.max(v1)
        i0 = jnp.min(jnp.where(v0 == m0, acci, big))
        i1 = jnp.min(jnp.where(v1 == m1, acci, big))
        stat_ref[1] = jnp.maximum(i0, i1)


def _scatter_body(s_ref, x_ref, cv_ref, y_ref):
    u = s_ref[0]
    aug = s_ref[1]
    x = x_ref[...]
    lane = lax.broadcasted_iota(jnp.int32, x.shape, 1)
    sub = lax.broadcasted_iota(jnp.int32, x.shape, 0)
    lp = u % 128
    s0 = ((u // 128) % 4) * 2
    sel = (lane == lp) & ((sub == s0) | (sub == s0 + 1)) & (aug != 0)
    val = jnp.where(sub == s0, cv_ref[0], cv_ref[1])
    y_ref[...] = jnp.where(sel, val, x)


def kernel(filter_by, current_values, augment):
    f, c = filter_by.shape
    r = (f * c) // _LANES
    w = (filter_by.reshape(r // 8, 4, _LANES, 2)
         .transpose(0, 1, 3, 2)
         .reshape(r, _LANES))

    y, stats = pl.pallas_call(
        _pass_body,
        grid=(r // _BR,),
        in_specs=[
            pl.BlockSpec(memory_space=pltpu.SMEM),
            pl.BlockSpec((_BR, _LANES), lambda i: (i, 0)),
        ],
        out_specs=[
            pl.BlockSpec((_BR, _LANES), lambda i: (i, 0)),
            pl.BlockSpec(memory_space=pltpu.SMEM),
        ],
        out_shape=[
            jax.ShapeDtypeStruct((r, _LANES), jnp.float32),
            jax.ShapeDtypeStruct((2,), jnp.int32),
        ],
        scratch_shapes=[
            pltpu.VMEM((_SUB, _LANES), jnp.float32),
            pltpu.VMEM((_SUB, _LANES), jnp.float32),
            pltpu.VMEM((_SUB, _LANES), jnp.int32),
        ],
    )(current_values, w)

    scal = jnp.stack([stats[1], augment.astype(jnp.int32)])
    y2 = pl.pallas_call(
        _scatter_body,
        grid_spec=pltpu.PrefetchScalarGridSpec(
            num_scalar_prefetch=1,
            grid=(1,),
            in_specs=[
                pl.BlockSpec((8, _LANES), lambda i, s: (s[0] // 512, 0)),
                pl.BlockSpec(memory_space=pltpu.SMEM),
            ],
            out_specs=pl.BlockSpec((8, _LANES), lambda i, s: (s[0] // 512, 0)),
        ),
        out_shape=jax.ShapeDtypeStruct((r, _LANES), jnp.float32),
        input_output_aliases={1: 0},
    )(scal, y, current_values)

    out = (y2.reshape(r // 8, 4, 2, _LANES)
           .transpose(0, 1, 3, 2)
           .reshape(f, c))
    return stats[0] == 0, out


# speedup vs baseline: 253.9575x; 1.0081x over previous
import jax
import jax.numpy as jnp
from jax import lax
from jax.experimental import pallas as pl
from jax.experimental.pallas import tpu as pltpu

_LANES = 128
_BR = 16384
_SUB = 8


def _pass_body(cv_ref, x_ref, y_ref, stat_ref, accd_ref, accv_ref, acci_ref):
    i = pl.program_id(0)
    n = pl.num_programs(0)

    shp = (_SUB, _LANES)
    lane = lax.broadcasted_iota(jnp.int32, shp, 1)
    sub = lax.broadcasted_iota(jnp.int32, shp, 0)
    oddsub = (sub & 1) == 1
    cvb = jnp.where(oddsub, cv_ref[1], cv_ref[0])
    idx0 = ((sub >> 1) << 7) | lane

    pos = jnp.float32(jnp.inf)
    neg = jnp.float32(-jnp.inf)

    @pl.when(i == 0)
    def _init():
        accd_ref[...] = jnp.full(shp, pos)
        accv_ref[...] = jnp.full(shp, neg)
        acci_ref[...] = jnp.zeros(shp, jnp.int32)

    accd = accd_ref[...]
    accv = accv_ref[...]
    acci = acci_ref[...]

    base = i * (_BR // _SUB) * 512
    for j in range(_BR // _SUB):
        x = x_ref[j * _SUB:(j + 1) * _SUB, :]
        y_ref[j * _SUB:(j + 1) * _SUB, :] = x
        d = x - cvb
        pm = jnp.maximum(d, pltpu.roll(d, _SUB - 1, 0))
        accd = jnp.minimum(accd, pm)
        gt = x > accv
        accv = jnp.where(gt, x, accv)
        acci = jnp.where(gt, idx0 + (base + j * 512), acci)

    accd_ref[...] = accd
    accv_ref[...] = accv
    acci_ref[...] = acci

    @pl.when(i == n - 1)
    def _fin():
        big = jnp.int32(2147483647)
        dmin = jnp.min(jnp.where(oddsub, pos, accd))
        stat_ref[0] = jnp.where(dmin <= 0.0, jnp.int32(1), jnp.int32(0))
        v0 = jnp.where(oddsub, neg, accv)
        v1 = jnp.where(oddsub, accv, neg)
        m0 = jnp.max(v0)
        m1 = jnp.max(v1)
        i0 = jnp.min(jnp.where(v0 == m0, acci, big))
        i1 = jnp.min(jnp.where(v1 == m1, acci, big))
        stat_ref[1] = jnp.maximum(i0, i1)


def _scatter_body(s_ref, x_ref, cv_ref, y_ref):
    u = s_ref[0]
    aug = s_ref[1]
    x = x_ref[...]
    lane = lax.broadcasted_iota(jnp.int32, x.shape, 1)
    sub = lax.broadcasted_iota(jnp.int32, x.shape, 0)
    lp = u % 128
    s0 = ((u // 128) % 4) * 2
    sel = (lane == lp) & ((sub == s0) | (sub == s0 + 1)) & (aug != 0)
    val = jnp.where(sub == s0, cv_ref[0], cv_ref[1])
    y_ref[...] = jnp.where(sel, val, x)


def kernel(filter_by, current_values, augment):
    f, c = filter_by.shape
    r = (f * c) // _LANES
    w = (filter_by.reshape(r // 8, 4, _LANES, 2)
         .transpose(0, 1, 3, 2)
         .reshape(r, _LANES))

    y, stats = pl.pallas_call(
        _pass_body,
        grid=(r // _BR,),
        in_specs=[
            pl.BlockSpec(memory_space=pltpu.SMEM),
            pl.BlockSpec((_BR, _LANES), lambda i: (i, 0)),
        ],
        out_specs=[
            pl.BlockSpec((_BR, _LANES), lambda i: (i, 0)),
            pl.BlockSpec(memory_space=pltpu.SMEM),
        ],
        out_shape=[
            jax.ShapeDtypeStruct((r, _LANES), jnp.float32),
            jax.ShapeDtypeStruct((2,), jnp.int32),
        ],
        scratch_shapes=[
            pltpu.VMEM((_SUB, _LANES), jnp.float32),
            pltpu.VMEM((_SUB, _LANES), jnp.float32),
            pltpu.VMEM((_SUB, _LANES), jnp.int32),
        ],
    )(current_values, w)

    scal = jnp.stack([stats[1], augment.astype(jnp.int32)])
    y2 = pl.pallas_call(
        _scatter_body,
        grid_spec=pltpu.PrefetchScalarGridSpec(
            num_scalar_prefetch=1,
            grid=(1,),
            in_specs=[
                pl.BlockSpec((8, _LANES), lambda i, s: (s[0] // 512, 0)),
                pl.BlockSpec(memory_space=pltpu.SMEM),
            ],
            out_specs=pl.BlockSpec((8, _LANES), lambda i, s: (s[0] // 512, 0)),
        ),
        out_shape=jax.ShapeDtypeStruct((r, _LANES), jnp.float32),
        input_output_aliases={1: 0},
    )(scal, y, current_values)

    out = (y2.reshape(r // 8, 4, 2, _LANES)
           .transpose(0, 1, 3, 2)
           .reshape(f, c))
    return stats[0] == 0, out


# speedup vs baseline: 268.0356x; 1.0554x over previous
import jax
import jax.numpy as jnp
from jax import lax
from jax.experimental import pallas as pl
from jax.experimental.pallas import tpu as pltpu

_LANES = 128
_BR = 8192
_SUB = 8


def _body(cv_ref, aug_ref, x_ref, y_hbm, stat_ref,
          ob_ref, tb_ref, accd_ref, accv_ref, acci_ref, sem, sem2):
    i = pl.program_id(0)
    n = pl.num_programs(0)
    slot = lax.rem(i, 2)

    shp = (_SUB, _LANES)
    lane = lax.broadcasted_iota(jnp.int32, shp, 1)
    sub = lax.broadcasted_iota(jnp.int32, shp, 0)
    oddsub = (sub & 1) == 1
    cvb = jnp.where(oddsub, cv_ref[1], cv_ref[0])
    idx0 = ((sub >> 1) << 7) | lane

    pos = jnp.float32(jnp.inf)
    neg = jnp.float32(-jnp.inf)

    @pl.when(i == 0)
    def _init():
        accd_ref[...] = jnp.full(shp, pos)
        accv_ref[...] = jnp.full(shp, neg)
        acci_ref[...] = jnp.zeros(shp, jnp.int32)

    @pl.when(i >= 2)
    def _drain():
        pltpu.make_async_copy(
            ob_ref.at[slot],
            y_hbm.at[pl.ds((i - 2) * _BR, _BR), :],
            sem.at[slot],
        ).wait()

    accd = accd_ref[...]
    accv = accv_ref[...]
    acci = acci_ref[...]

    base = i * (_BR // _SUB) * 512
    for j in range(_BR // _SUB):
        x = x_ref[j * _SUB:(j + 1) * _SUB, :]
        ob_ref[slot, j * _SUB:(j + 1) * _SUB, :] = x
        d = x - cvb
        pm = jnp.maximum(d, pltpu.roll(d, _SUB - 1, 0))
        accd = jnp.minimum(accd, pm)
        gt = x > accv
        accv = jnp.where(gt, x, accv)
        acci = jnp.where(gt, idx0 + (base + j * 512), acci)

    accd_ref[...] = accd
    accv_ref[...] = accv
    acci_ref[...] = acci

    pltpu.make_async_copy(
        ob_ref.at[slot],
        y_hbm.at[pl.ds(i * _BR, _BR), :],
        sem.at[slot],
    ).start()

    @pl.when(i == n - 1)
    def _fin():
        pltpu.make_async_copy(
            ob_ref.at[1 - slot],
            y_hbm.at[pl.ds((i - 1) * _BR, _BR), :],
            sem.at[1 - slot],
        ).wait()
        pltpu.make_async_copy(
            ob_ref.at[slot],
            y_hbm.at[pl.ds(i * _BR, _BR), :],
            sem.at[slot],
        ).wait()

        big = jnp.int32(2147483647)
        dmin = jnp.min(jnp.where(oddsub, pos, accd_ref[...]))
        stat_ref[0] = jnp.where(dmin <= 0.0, jnp.int32(1), jnp.int32(0))
        v0 = jnp.where(oddsub, neg, accv_ref[...])
        v1 = jnp.where(oddsub, accv_ref[...], neg)
        m0 = jnp.max(v0)
        m1 = jnp.max(v1)
        i0 = jnp.min(jnp.where(v0 == m0, acci_ref[...], big))
        i1 = jnp.min(jnp.where(v1 == m1, acci_ref[...], big))
        u = jnp.maximum(i0, i1)
        stat_ref[1] = u

        base8 = pl.multiple_of((u // 512) * 8, 8)
        rd = pltpu.make_async_copy(
            y_hbm.at[pl.ds(base8, 8), :], tb_ref, sem2)
        rd.start()
        rd.wait()
        lp = u % 128
        s0 = ((u // 128) % 4) * 2
        sel = ((lane == lp) & ((sub == s0) | (sub == s0 + 1))
               & (aug_ref[0] != 0))
        val = jnp.where(sub == s0, cv_ref[0], cv_ref[1])
        tb_ref[...] = jnp.where(sel, val, tb_ref[...])
        wr = pltpu.make_async_copy(
            tb_ref, y_hbm.at[pl.ds(base8, 8), :], sem2)
        wr.start()
        wr.wait()


def kernel(filter_by, current_values, augment):
    f, c = filter_by.shape
    r = (f * c) // _LANES
    w = (filter_by.reshape(r // 8, 4, _LANES, 2)
         .transpose(0, 1, 3, 2)
         .reshape(r, _LANES))

    y, stats = pl.pallas_call(
        _body,
        grid=(r // _BR,),
        in_specs=[
            pl.BlockSpec(memory_space=pltpu.SMEM),
            pl.BlockSpec(memory_space=pltpu.SMEM),
            pl.BlockSpec((_BR, _LANES), lambda i: (i, 0)),
        ],
        out_specs=[
            pl.BlockSpec(memory_space=pl.ANY),
            pl.BlockSpec(memory_space=pltpu.SMEM),
        ],
        out_shape=[
            jax.ShapeDtypeStruct((r, _LANES), jnp.float32),
            jax.ShapeDtypeStruct((2,), jnp.int32),
        ],
        scratch_shapes=[
            pltpu.VMEM((2, _BR, _LANES), jnp.float32),
            pltpu.VMEM((_SUB, _LANES), jnp.float32),
            pltpu.VMEM((_SUB, _LANES), jnp.float32),
            pltpu.VMEM((_SUB, _LANES), jnp.float32),
            pltpu.VMEM((_SUB, _LANES), jnp.int32),
            pltpu.SemaphoreType.DMA((2,)),
            pltpu.SemaphoreType.DMA,
        ],
    )(current_values, augment.astype(jnp.int32).reshape(1), w)

    out = (y.reshape(r // 8, 4, 2, _LANES)
           .transpose(0, 1, 3, 2)
           .reshape(f, c))
    return stats[0] == 0, out


# speedup vs baseline: 272.6103x; 1.0171x over previous
import jax
import jax.numpy as jnp
from jax import lax
from jax.experimental import pallas as pl
from jax.experimental.pallas import tpu as pltpu

_LANES = 128
_BR = 16384
_SUB = 8


def _body(cv_ref, aug_ref, x_ref, y_hbm, stat_ref,
          ob_ref, tb_ref, accd_ref, accv_ref, acci_ref, sem, sem2):
    i = pl.program_id(0)
    n = pl.num_programs(0)
    slot = lax.rem(i, 2)

    shp = (_SUB, _LANES)
    lane = lax.broadcasted_iota(jnp.int32, shp, 1)
    sub = lax.broadcasted_iota(jnp.int32, shp, 0)
    oddsub = (sub & 1) == 1
    cvb = jnp.where(oddsub, cv_ref[1], cv_ref[0])
    idx0 = ((sub >> 1) << 7) | lane

    pos = jnp.float32(jnp.inf)
    neg = jnp.float32(-jnp.inf)

    @pl.when(i == 0)
    def _init():
        accd_ref[...] = jnp.full(shp, pos)
        accv_ref[...] = jnp.full(shp, neg)
        acci_ref[...] = jnp.zeros(shp, jnp.int32)

    @pl.when(i >= 2)
    def _drain():
        pltpu.make_async_copy(
            ob_ref.at[slot],
            y_hbm.at[pl.ds((i - 2) * _BR, _BR), :],
            sem.at[slot],
        ).wait()

    accd = accd_ref[...]
    accv = accv_ref[...]
    acci = acci_ref[...]

    base = i * (_BR // _SUB) * 512
    for j in range(_BR // _SUB):
        x = x_ref[j * _SUB:(j + 1) * _SUB, :]
        ob_ref[slot, j * _SUB:(j + 1) * _SUB, :] = x
        d = x - cvb
        pm = jnp.maximum(d, pltpu.roll(d, _SUB - 1, 0))
        accd = jnp.minimum(accd, pm)
        gt = x > accv
        accv = jnp.where(gt, x, accv)
        acci = jnp.where(gt, idx0 + (base + j * 512), acci)

    accd_ref[...] = accd
    accv_ref[...] = accv
    acci_ref[...] = acci

    pltpu.make_async_copy(
        ob_ref.at[slot],
        y_hbm.at[pl.ds(i * _BR, _BR), :],
        sem.at[slot],
    ).start()

    @pl.when(i == n - 1)
    def _fin():
        pltpu.make_async_copy(
            ob_ref.at[1 - slot],
            y_hbm.at[pl.ds((i - 1) * _BR, _BR), :],
            sem.at[1 - slot],
        ).wait()
        pltpu.make_async_copy(
            ob_ref.at[slot],
            y_hbm.at[pl.ds(i * _BR, _BR), :],
            sem.at[slot],
        ).wait()

        big = jnp.int32(2147483647)
        dmin = jnp.min(jnp.where(oddsub, pos, accd_ref[...]))
        stat_ref[0] = jnp.where(dmin <= 0.0, jnp.int32(1), jnp.int32(0))
        v0 = jnp.where(oddsub, neg, accv_ref[...])
        v1 = jnp.where(oddsub, accv_ref[...], neg)
        m0 = jnp.max(v0)
        m1 = jnp.max(v1)
        i0 = jnp.min(jnp.where(v0 == m0, acci_ref[...], big))
        i1 = jnp.min(jnp.where(v1 == m1, acci_ref[...], big))
        u = jnp.maximum(i0, i1)
        stat_ref[1] = u

        base8 = pl.multiple_of((u // 512) * 8, 8)
        rd = pltpu.make_async_copy(
            y_hbm.at[pl.ds(base8, 8), :], tb_ref, sem2)
        rd.start()
        rd.wait()
        lp = u % 128
        s0 = ((u // 128) % 4) * 2
        sel = ((lane == lp) & ((sub == s0) | (sub == s0 + 1))
               & (aug_ref[0] != 0))
        val = jnp.where(sub == s0, cv_ref[0], cv_ref[1])
        tb_ref[...] = jnp.where(sel, val, tb_ref[...])
        wr = pltpu.make_async_copy(
            tb_ref, y_hbm.at[pl.ds(base8, 8), :], sem2)
        wr.start()
        wr.wait()


def kernel(filter_by, current_values, augment):
    f, c = filter_by.shape
    r = (f * c) // _LANES
    w = (filter_by.reshape(r // 8, 4, _LANES, 2)
         .transpose(0, 1, 3, 2)
         .reshape(r, _LANES))

    y, stats = pl.pallas_call(
        _body,
        grid=(r // _BR,),
        in_specs=[
            pl.BlockSpec(memory_space=pltpu.SMEM),
            pl.BlockSpec(memory_space=pltpu.SMEM),
            pl.BlockSpec((_BR, _LANES), lambda i: (i, 0)),
        ],
        out_specs=[
            pl.BlockSpec(memory_space=pl.ANY),
            pl.BlockSpec(memory_space=pltpu.SMEM),
        ],
        out_shape=[
            jax.ShapeDtypeStruct((r, _LANES), jnp.float32),
            jax.ShapeDtypeStruct((2,), jnp.int32),
        ],
        scratch_shapes=[
            pltpu.VMEM((2, _BR, _LANES), jnp.float32),
            pltpu.VMEM((_SUB, _LANES), jnp.float32),
            pltpu.VMEM((_SUB, _LANES), jnp.float32),
            pltpu.VMEM((_SUB, _LANES), jnp.float32),
            pltpu.VMEM((_SUB, _LANES), jnp.int32),
            pltpu.SemaphoreType.DMA((2,)),
            pltpu.SemaphoreType.DMA,
        ],
    )(current_values, augment.astype(jnp.int32).reshape(1), w)

    out = (y.reshape(r // 8, 4, 2, _LANES)
           .transpose(0, 1, 3, 2)
           .reshape(f, c))
    return stats[0] == 0, out


# speedup vs baseline: 294.2941x; 1.0795x over previous
import jax
import jax.numpy as jnp
from jax import lax
from jax.experimental import pallas as pl
from jax.experimental.pallas import tpu as pltpu

_LANES = 128
_BR = 16384
_SUB = 8


def _body(cv_ref, aug_ref, x_ref, y_hbm, stat_ref,
          ob_ref, tb_ref, accd_ref, accv_ref, acci_ref, sem, sem2):
    i = pl.program_id(0)
    n = pl.num_programs(0)
    slot = lax.rem(i, 2)

    shp = (_SUB, _LANES)
    lane = lax.broadcasted_iota(jnp.int32, shp, 1)
    sub = lax.broadcasted_iota(jnp.int32, shp, 0)
    oddsub = (sub & 1) == 1
    cvb = jnp.where(oddsub, cv_ref[1], cv_ref[0])
    idx0 = ((sub >> 1) << 7) | lane

    pos = jnp.float32(jnp.inf)
    neg = jnp.float32(-jnp.inf)

    @pl.when(i == 0)
    def _init():
        accd_ref[...] = jnp.full(shp, pos)
        accv_ref[...] = jnp.full(shp, neg)
        acci_ref[...] = jnp.zeros(shp, jnp.int32)

    _H = _BR // 2

    def _flush_dma(s, blk, h):
        return pltpu.make_async_copy(
            ob_ref.at[s, pl.ds(h * _H, _H)],
            y_hbm.at[pl.ds(blk * _BR + h * _H, _H), :],
            sem.at[s, h],
        )

    @pl.when(i >= 2)
    def _drain():
        _flush_dma(slot, i - 2, 0).wait()
        _flush_dma(slot, i - 2, 1).wait()

    accd = accd_ref[...]
    accv = accv_ref[...]
    acci = acci_ref[...]

    base = i * (_BR // _SUB) * 512
    for j in range(_BR // _SUB):
        x = x_ref[j * _SUB:(j + 1) * _SUB, :]
        ob_ref[slot, j * _SUB:(j + 1) * _SUB, :] = x
        d = x - cvb
        pm = jnp.maximum(d, pltpu.roll(d, _SUB - 1, 0))
        accd = jnp.minimum(accd, pm)
        gt = x > accv
        accv = jnp.where(gt, x, accv)
        acci = jnp.where(gt, idx0 + (base + j * 512), acci)

    accd_ref[...] = accd
    accv_ref[...] = accv
    acci_ref[...] = acci

    _flush_dma(slot, i, 0).start()
    _flush_dma(slot, i, 1).start()

    @pl.when(i == n - 1)
    def _fin():
        _flush_dma(1 - slot, i - 1, 0).wait()
        _flush_dma(1 - slot, i - 1, 1).wait()
        _flush_dma(slot, i, 0).wait()
        _flush_dma(slot, i, 1).wait()

        big = jnp.int32(2147483647)
        dmin = jnp.min(jnp.where(oddsub, pos, accd_ref[...]))
        stat_ref[0] = jnp.where(dmin <= 0.0, jnp.int32(1), jnp.int32(0))
        v0 = jnp.where(oddsub, neg, accv_ref[...])
        v1 = jnp.where(oddsub, accv_ref[...], neg)
        m0 = jnp.max(v0)
        m1 = jnp.max(v1)
        i0 = jnp.min(jnp.where(v0 == m0, acci_ref[...], big))
        i1 = jnp.min(jnp.where(v1 == m1, acci_ref[...], big))
        u = jnp.maximum(i0, i1)
        stat_ref[1] = u

        base8 = pl.multiple_of((u // 512) * 8, 8)
        rd = pltpu.make_async_copy(
            y_hbm.at[pl.ds(base8, 8), :], tb_ref, sem2)
        rd.start()
        rd.wait()
        lp = u % 128
        s0 = ((u // 128) % 4) * 2
        sel = ((lane == lp) & ((sub == s0) | (sub == s0 + 1))
               & (aug_ref[0] != 0))
        val = jnp.where(sub == s0, cv_ref[0], cv_ref[1])
        tb_ref[...] = jnp.where(sel, val, tb_ref[...])
        wr = pltpu.make_async_copy(
            tb_ref, y_hbm.at[pl.ds(base8, 8), :], sem2)
        wr.start()
        wr.wait()


def kernel(filter_by, current_values, augment):
    f, c = filter_by.shape
    r = (f * c) // _LANES
    w = (filter_by.reshape(r // 8, 4, _LANES, 2)
         .transpose(0, 1, 3, 2)
         .reshape(r, _LANES))

    y, stats = pl.pallas_call(
        _body,
        grid=(r // _BR,),
        in_specs=[
            pl.BlockSpec(memory_space=pltpu.SMEM),
            pl.BlockSpec(memory_space=pltpu.SMEM),
            pl.BlockSpec((_BR, _LANES), lambda i: (i, 0)),
        ],
        out_specs=[
            pl.BlockSpec(memory_space=pl.ANY),
            pl.BlockSpec(memory_space=pltpu.SMEM),
        ],
        out_shape=[
            jax.ShapeDtypeStruct((r, _LANES), jnp.float32),
            jax.ShapeDtypeStruct((2,), jnp.int32),
        ],
        scratch_shapes=[
            pltpu.VMEM((2, _BR, _LANES), jnp.float32),
            pltpu.VMEM((_SUB, _LANES), jnp.float32),
            pltpu.VMEM((_SUB, _LANES), jnp.float32),
            pltpu.VMEM((_SUB, _LANES), jnp.float32),
            pltpu.VMEM((_SUB, _LANES), jnp.int32),
            pltpu.SemaphoreType.DMA((2, 2)),
            pltpu.SemaphoreType.DMA,
        ],
    )(current_values, augment.astype(jnp.int32).reshape(1), w)

    out = (y.reshape(r // 8, 4, 2, _LANES)
           .transpose(0, 1, 3, 2)
           .reshape(f, c))
    return stats[0] == 0, out


# speedup vs baseline: 294.7103x; 1.0014x over previous
import jax
import jax.numpy as jnp
from jax import lax
from jax.experimental import pallas as pl
from jax.experimental.pallas import tpu as pltpu

_LANES = 128
_BR = 16384
_SUB = 8
_NW = 4


def _body(cv_ref, aug_ref, x_ref, y_hbm, stat_ref,
          ob_ref, tb_ref, accd_ref, accv_ref, acci_ref, sem, sem2):
    i = pl.program_id(0)
    n = pl.num_programs(0)
    slot = lax.rem(i, 2)

    shp = (_SUB, _LANES)
    lane = lax.broadcasted_iota(jnp.int32, shp, 1)
    sub = lax.broadcasted_iota(jnp.int32, shp, 0)
    oddsub = (sub & 1) == 1
    cvb = jnp.where(oddsub, cv_ref[1], cv_ref[0])
    idx0 = ((sub >> 1) << 7) | lane

    pos = jnp.float32(jnp.inf)
    neg = jnp.float32(-jnp.inf)

    @pl.when(i == 0)
    def _init():
        accd_ref[...] = jnp.full(shp, pos)
        accv_ref[...] = jnp.full(shp, neg)
        acci_ref[...] = jnp.zeros(shp, jnp.int32)

    _H = _BR // _NW

    def _flush_dma(s, blk, h):
        return pltpu.make_async_copy(
            ob_ref.at[s, pl.ds(h * _H, _H)],
            y_hbm.at[pl.ds(blk * _BR + h * _H, _H), :],
            sem.at[s, h],
        )

    @pl.when(i >= 2)
    def _drain():
        for h in range(_NW):
            _flush_dma(slot, i - 2, h).wait()

    accd = accd_ref[...]
    accv = accv_ref[...]
    acci = acci_ref[...]

    base = i * (_BR // _SUB) * 512
    for j in range(_BR // _SUB):
        x = x_ref[j * _SUB:(j + 1) * _SUB, :]
        ob_ref[slot, j * _SUB:(j + 1) * _SUB, :] = x
        d = x - cvb
        pm = jnp.maximum(d, pltpu.roll(d, _SUB - 1, 0))
        accd = jnp.minimum(accd, pm)
        gt = x > accv
        accv = jnp.where(gt, x, accv)
        acci = jnp.where(gt, idx0 + (base + j * 512), acci)

    accd_ref[...] = accd
    accv_ref[...] = accv
    acci_ref[...] = acci

    for h in range(_NW):
        _flush_dma(slot, i, h).start()

    @pl.when(i == n - 1)
    def _fin():
        for h in range(_NW):
            _flush_dma(1 - slot, i - 1, h).wait()
        for h in range(_NW):
            _flush_dma(slot, i, h).wait()

        big = jnp.int32(2147483647)
        dmin = jnp.min(jnp.where(oddsub, pos, accd_ref[...]))
        stat_ref[0] = jnp.where(dmin <= 0.0, jnp.int32(1), jnp.int32(0))
        v0 = jnp.where(oddsub, neg, accv_ref[...])
        v1 = jnp.where(oddsub, accv_ref[...], neg)
        m0 = jnp.max(v0)
        m1 = jnp.max(v1)
        i0 = jnp.min(jnp.where(v0 == m0, acci_ref[...], big))
        i1 = jnp.min(jnp.where(v1 == m1, acci_ref[...], big))
        u = jnp.maximum(i0, i1)
        stat_ref[1] = u

        base8 = pl.multiple_of((u // 512) * 8, 8)
        rd = pltpu.make_async_copy(
            y_hbm.at[pl.ds(base8, 8), :], tb_ref, sem2)
        rd.start()
        rd.wait()
        lp = u % 128
        s0 = ((u // 128) % 4) * 2
        sel = ((lane == lp) & ((sub == s0) | (sub == s0 + 1))
               & (aug_ref[0] != 0))
        val = jnp.where(sub == s0, cv_ref[0], cv_ref[1])
        tb_ref[...] = jnp.where(sel, val, tb_ref[...])
        wr = pltpu.make_async_copy(
            tb_ref, y_hbm.at[pl.ds(base8, 8), :], sem2)
        wr.start()
        wr.wait()


def kernel(filter_by, current_values, augment):
    f, c = filter_by.shape
    r = (f * c) // _LANES
    w = (filter_by.reshape(r // 8, 4, _LANES, 2)
         .transpose(0, 1, 3, 2)
         .reshape(r, _LANES))

    y, stats = pl.pallas_call(
        _body,
        grid=(r // _BR,),
        in_specs=[
            pl.BlockSpec(memory_space=pltpu.SMEM),
            pl.BlockSpec(memory_space=pltpu.SMEM),
            pl.BlockSpec((_BR, _LANES), lambda i: (i, 0)),
        ],
        out_specs=[
            pl.BlockSpec(memory_space=pl.ANY),
            pl.BlockSpec(memory_space=pltpu.SMEM),
        ],
        out_shape=[
            jax.ShapeDtypeStruct((r, _LANES), jnp.float32),
            jax.ShapeDtypeStruct((2,), jnp.int32),
        ],
        scratch_shapes=[
            pltpu.VMEM((2, _BR, _LANES), jnp.float32),
            pltpu.VMEM((_SUB, _LANES), jnp.float32),
            pltpu.VMEM((_SUB, _LANES), jnp.float32),
            pltpu.VMEM((_SUB, _LANES), jnp.float32),
            pltpu.VMEM((_SUB, _LANES), jnp.int32),
            pltpu.SemaphoreType.DMA((2, _NW)),
            pltpu.SemaphoreType.DMA,
        ],
    )(current_values, augment.astype(jnp.int32).reshape(1), w)

    out = (y.reshape(r // 8, 4, 2, _LANES)
           .transpose(0, 1, 3, 2)
           .reshape(f, c))
    return stats[0] == 0, out
